# Initial kernel scaffold; baseline (speedup 1.0000x reference)
import jax
import jax.numpy as jnp
from jax.experimental import pallas as pl


def kernel(v, s, edge_index, d_ij, dir_ij, W1, b1, W2, b2, W3, b3):
    raise NotImplementedError("write your pallas kernel here")


# speedup vs baseline: 7.0697x; 7.0697x over previous
import functools

import jax
import jax.numpy as jnp
from jax import lax
from jax.experimental import pallas as pl
from jax.experimental.pallas import tpu as pltpu
from jax.experimental.pallas import tpu_sc as plsc

H = 128
N = 10000
E = 320000
NW = 32
EPW = E // NW
BLK = 64
NPB = EPW // BLK + 3
NPAD = 10240
ACC_R = 1280
NROUND = NPAD // ACC_R
NPH = 5

_HI = jax.lax.Precision.HIGHEST
_SC_PARAMS = pltpu.CompilerParams(needs_layout_passes=False,
                                  use_tc_tiling_on_sc=False)


def _mlp_body(s_ref, w1_ref, b1_ref, w2_ref, b2_ref, v_ref,
              t_ref, t0_ref, t1_ref, t2_ref):
    h = jnp.tanh(
        lax.dot_general(s_ref[...], w1_ref[...], (((1,), (1,)), ((), ())),
                        precision=_HI) + b1_ref[0:1, :])
    se = lax.dot_general(h, w2_ref[...], (((1,), (1,)), ((), ())),
                         precision=_HI) + b2_ref[0:1, :]
    a = se[:, 0:H]
    r = se[:, 2 * H:3 * H]
    t_ref[...] = se[:, H:2 * H]
    t0_ref[...] = jnp.concatenate([a * v_ref[:, 0, :], r], axis=1)
    t1_ref[...] = jnp.concatenate([a * v_ref[:, 1, :], r], axis=1)
    t2_ref[...] = jnp.concatenate([a * v_ref[:, 2, :], r], axis=1)


def _mlp_tables(s_pad, W1, b1, W2, b2, v_pad):
    blk = 512
    grid = NPAD // blk
    out1 = jax.ShapeDtypeStruct((NPAD, H), jnp.float32)
    out2 = jax.ShapeDtypeStruct((NPAD, 2 * H), jnp.float32)
    return pl.pallas_call(
        _mlp_body,
        grid=(grid,),
        in_specs=[
            pl.BlockSpec((blk, H), lambda i: (i, 0)),
            pl.BlockSpec((H, H), lambda i: (0, 0)),
            pl.BlockSpec((8, H), lambda i: (0, 0)),
            pl.BlockSpec((3 * H, H), lambda i: (0, 0)),
            pl.BlockSpec((8, 3 * H), lambda i: (0, 0)),
            pl.BlockSpec((blk, 3, H), lambda i: (i, 0, 0)),
        ],
        out_specs=[pl.BlockSpec((blk, H), lambda i: (i, 0))]
        + [pl.BlockSpec((blk, 2 * H), lambda i: (i, 0))] * 3,
        out_shape=[out1, out2, out2, out2],
    )(s_pad, W1, jnp.broadcast_to(b1, (8, H)), W2,
      jnp.broadcast_to(b2, (8, 3 * H)), v_pad)


_MESH = plsc.VectorSubcoreMesh(core_axis_name="c", subcore_axis_name="s")


def _edge_phases(S_t, T0_t, T1_t, T2_t, edge32, d_ij, dir_ij, w3, b3):
    @functools.partial(
        pl.kernel,
        out_type=jax.ShapeDtypeStruct((2, NPH, NPAD, H), jnp.float32),
        mesh=_MESH,
        compiler_params=_SC_PARAMS,
        scratch_types=[
            pltpu.VMEM((2, EPW + 16), jnp.int32),
            pltpu.VMEM((EPW + 16,), jnp.float32),
            pltpu.VMEM((3 * (EPW + 16),), jnp.float32),
            pltpu.VMEM((NPB * BLK,), jnp.int32),
            pltpu.VMEM((2, BLK), jnp.int32),
            pltpu.VMEM((2, BLK), jnp.int32),
            pltpu.VMEM((2, BLK), jnp.float32),
            pltpu.VMEM((2, BLK), jnp.float32),
            pltpu.VMEM((BLK, 2 * H), jnp.float32),
            pltpu.VMEM((BLK, 2 * H), jnp.float32),
            pltpu.VMEM((BLK, H), jnp.float32),
            pltpu.VMEM((BLK, H), jnp.float32),
            pltpu.VMEM((3 * H,), jnp.float32),
            pltpu.VMEM((3 * H,), jnp.float32),
            pltpu.SemaphoreType.DMA,
            pltpu.SemaphoreType.DMA,
            pltpu.VMEM_SHARED((ACC_R + 8, H), jnp.float32),
        ],
    )
    def k(S_hbm, T0_hbm, T1_hbm, T2_hbm, e_hbm,
          d_hbm, dir_hbm, w3_hbm, b3_hbm, part_hbm,
          e_v, d_v, dir_v, eidl_v, src_blk, dloc_blk, dd_blk, kk_blk,
          t_buf0, t_buf1, out_buf0, out_buf1, w3_v, b3_v,
          gsem, ssem, accum):
        c = lax.axis_index("c")
        sid = lax.axis_index("s")
        wid = sid * 2 + c
        rps = ACC_R // 16
        stripe = pl.ds(sid * rps, rps)
        t_bufs = (t_buf0, t_buf1)
        out_bufs = (out_buf0, out_buf1)
        gsems = (gsem, gsem)
        ssems = (ssem, ssem)
        pltpu.sync_copy(e_hbm.at[:, pl.ds(wid * EPW, EPW)],
                        e_v.at[:, pl.ds(0, EPW)])
        pltpu.sync_copy(d_hbm.at[pl.ds(wid * EPW, EPW)],
                        d_v.at[pl.ds(0, EPW)])
        pltpu.sync_copy(dir_hbm.at[pl.ds(wid * 3 * EPW, 3 * EPW)],
                        dir_v.at[pl.ds(0, 3 * EPW)])
        pltpu.sync_copy(w3_hbm, w3_v)
        pltpu.sync_copy(b3_hbm, b3_v)

        zero16 = jnp.zeros((16,), jnp.float32)
        z16i = jnp.zeros((16,), jnp.int32)
        o16i = jnp.ones((16,), jnp.int32)
        iota16 = lax.iota(jnp.int32, 16)
        sent16 = jnp.full((16,), EPW, jnp.int32)

        e_v[0, pl.ds(EPW, 16)] = z16i
        e_v[1, pl.ds(EPW, 16)] = jnp.full((16,), -1, jnp.int32)
        d_v[pl.ds(EPW, 16)] = zero16
        for t in range(3):
            dir_v[pl.ds(3 * EPW + 16 * t, 16)] = zero16

        def start_phase():
            @pl.loop(0, rps // 16)
            def _(t):
                pltpu.sync_copy(out_buf1.at[pl.ds(0, 16)],
                                accum.at[pl.ds(sid * rps + t * 16, 16)])

            @pl.when(sid == 15)
            def _():
                pltpu.sync_copy(out_buf1.at[pl.ds(0, 8)],
                                accum.at[pl.ds(ACC_R, 8)])
            plsc.subcore_barrier()

        def end_phase(p, base):
            plsc.subcore_barrier()
            pltpu.sync_copy(accum.at[stripe],
                            part_hbm.at[c, p, pl.ds(base + sid * rps, rps)])

        def build_block(b, h, base, p):
            p16 = jnp.full((16,), p, jnp.int32)
            for t in range(BLK // 16):
                eid16 = eidl_v[pl.ds(b * BLK + 16 * t, 16)]
                src_blk[h, pl.ds(16 * t, 16)] = plsc.load_gather(
                    e_v, [z16i, eid16])
                dl16 = plsc.load_gather(e_v, [o16i, eid16]) - base
                m = (dl16 >= 0) & (dl16 < ACC_R)
                dloc_blk[h, pl.ds(16 * t, 16)] = jnp.where(m, dl16, ACC_R)
                dd_blk[h, pl.ds(16 * t, 16)] = plsc.load_gather(d_v, [eid16])
                kk_blk[h, pl.ds(16 * t, 16)] = plsc.load_gather(
                    dir_v, [eid16 * 3 + p16])

        @pl.loop(0, NROUND)
        def _(r):
            base = r * ACC_R

            @pl.loop(0, (NPB * BLK) // 16)
            def _(t):
                eidl_v[pl.ds(16 * t, 16)] = sent16

            def cbody(t, off):
                dd = e_v[1, pl.ds(16 * t, 16)]
                ld = dd - base
                m = (ld >= 0) & (ld < ACC_R)
                cs = plsc.cumsum(m.astype(jnp.int32))
                plsc.store_scatter(eidl_v, [off + cs - 1], iota16 + 16 * t,
                                   mask=m)
                return off + cs[15]

            cnt = lax.fori_loop(0, EPW // 16, cbody, jnp.int32(0))
            nb2 = (cnt + 2 * BLK - 1) // (2 * BLK)

            for p, (tbl, tw) in enumerate(
                    ((S_hbm, H), (T0_hbm, 2 * H), (T1_hbm, 2 * H),
                     (T2_hbm, 2 * H))):
                start_phase()

                @pl.loop(0, nb2)
                def _(t2):
                    gh = []
                    for h in (0, 1):
                        build_block(2 * t2 + h, h, base, max(p - 1, 0))
                        gh.append(pltpu.async_copy(
                            tbl.at[src_blk.at[h]],
                            t_bufs[h].at[:, pl.ds(0, tw)], gsems[h]))
                    sh = []
                    for h in (0, 1):
                        gh[h].wait()
                        tb = t_bufs[h]
                        ob = out_bufs[h]
                        h16 = jnp.full((16,), h, jnp.int32)

                        @pl.loop(0, BLK)
                        def _(e):
                            e16 = jnp.full((16,), e, jnp.int32)
                            d16 = plsc.load_gather(dd_blk, [h16, e16])
                            if p == 0:
                                for f in range(H // 16):
                                    sl = pl.ds(16 * f, 16)
                                    ob[e, sl] = tb[e, sl] * (
                                        d16 * wv[f] + bv[f])
                            else:
                                k16 = plsc.load_gather(kk_blk, [h16, e16])
                                for f in range(H // 16):
                                    sl = pl.ds(16 * f, 16)
                                    slh = pl.ds(H + 16 * f, 16)
                                    ca = d16 * wv[f] + bv[f]
                                    cr = d16 * wr[f] + br[f]
                                    ob[e, sl] = ((tb[e, slh] * cr) * k16
                                                 + tb[e, sl] * ca)

                        sh.append(pltpu.async_copy(
                            ob, accum.at[dloc_blk.at[h]], ssems[h],
                            add=True))
                    for h in (0, 1):
                        sh[h].wait()

                end_phase(p, base)


    return k(S_t, T0_t, T1_t, T2_t, edge32, d_ij, dir_ij, w3, b3)


def _fin_body(v_ref, s_ref, p_ref, vo_ref, so_ref):
    cnt = p_ref[0, 4, :, 0:1] + p_ref[1, 4, :, 0:1]
    inv = 1.0 / jnp.maximum(cnt, 1.0)
    so_ref[...] = s_ref[...] + (p_ref[0, 0, :, 0:H] + p_ref[1, 0, :, 0:H]) * inv
    for kk in range(3):
        vo_ref[:, kk, :] = v_ref[:, kk, :] + (
            p_ref[0, kk + 1, :, 0:H] + p_ref[1, kk + 1, :, 0:H]) * inv


def _finalize(v, s, parts):
    blk = 400
    grid = N // blk
    return pl.pallas_call(
        _fin_body,
        grid=(grid,),
        in_specs=[
            pl.BlockSpec((blk, 3, H), lambda i: (i, 0, 0)),
            pl.BlockSpec((blk, H), lambda i: (i, 0)),
            pl.BlockSpec((2, NPH, blk, H), lambda i: (0, 0, i, 0)),
        ],
        out_specs=[
            pl.BlockSpec((blk, 3, H), lambda i: (i, 0, 0)),
            pl.BlockSpec((blk, H), lambda i: (i, 0)),
        ],
        out_shape=[
            jax.ShapeDtypeStruct((N, 3, H), jnp.float32),
            jax.ShapeDtypeStruct((N, H), jnp.float32),
        ],
    )(v, s, parts)


def kernel(v, s, edge_index, d_ij, dir_ij, W1, b1, W2, b2, W3, b3):
    if edge_index.dtype == jnp.int32:
        edge32 = edge_index
    else:
        edge32 = jax.lax.bitcast_convert_type(edge_index, jnp.int32)[:, :, 0]
    dflat = d_ij.reshape(E)
    dirflat = dir_ij.reshape(3 * E)
    w3 = W3[:, 0]

    s_pad = jnp.pad(s, ((0, NPAD - N), (0, 0)))
    v_pad = jnp.pad(v, ((0, NPAD - N), (0, 0), (0, 0)))
    S_t, T0_t, T1_t, T2_t = _mlp_tables(s_pad, W1, b1, W2, b2, v_pad)

    parts = _edge_phases(S_t, T0_t, T1_t, T2_t, edge32,
                         dflat, dirflat, w3, b3)

    return _finalize(v, s, parts)
